# baseline (device time: 237813 ns/iter reference)
import functools

import jax
import jax.numpy as jnp
from jax import lax
from jax.experimental import pallas as pl
from jax.experimental.pallas import tpu as pltpu

N_DEV = 4


def kernel(x, w_mat):
    m_loc, k = x.shape
    _, n_loc = w_mat.shape
    m_half = m_loc // 2
    m_q = m_half // 2
    m_glob = N_DEV * m_loc
    n_hops = N_DEV - 1
    n_sub = 2 * N_DEV * 2
    n_wp = k // m_q

    x = x.astype(jnp.bfloat16)
    w_mat = w_mat.astype(jnp.bfloat16)

    def body(
        x_ref, w_ref, out_ref,
        own_cw, own_ccw, comm_cw, comm_ccw, stage,
        send_cw, recv_cw, send_ccw, recv_ccw,
        stage_sems, store_sems, load_sems,
        amax_box, amax_recv, amax_send_sems, amax_recv_sems,
    ):
        me = lax.axis_index("i")
        right = lax.rem(me + 1, N_DEV)
        left = lax.rem(me + N_DEV - 1, N_DEV)

        bar = pltpu.get_barrier_semaphore()
        for off in range(1, N_DEV):
            peer = lax.rem(me + off, N_DEV)
            pl.semaphore_signal(
                bar, inc=1, device_id=(peer,),
                device_id_type=pl.DeviceIdType.MESH,
            )
        pl.semaphore_wait(bar, N_DEV - 1)

        def rdma(src, dst, sems, h, dev):
            return pltpu.make_async_remote_copy(
                src_ref=src, dst_ref=dst,
                send_sem=sems[0].at[h], recv_sem=sems[1].at[h],
                device_id=(dev,), device_id_type=pl.DeviceIdType.MESH,
            )

        cw_sems = (send_cw, recv_cw)
        ccw_sems = (send_ccw, recv_ccw)
        st_top = pltpu.make_async_copy(
            x_ref.at[pl.ds(0, m_half), :], own_cw, stage_sems.at[0]
        )
        st_bot = pltpu.make_async_copy(
            x_ref.at[pl.ds(m_half, m_half), :], own_ccw, stage_sems.at[1]
        )
        st_top.start()
        st_bot.start()
        st_top.wait()
        st_bot.wait()

        cw = [
            rdma(own_cw, comm_cw.at[0], cw_sems, 0, right),
            rdma(comm_cw.at[0], comm_cw.at[1], cw_sems, 1, right),
            rdma(comm_cw.at[1, pl.ds(0, m_q), :],
                 comm_cw.at[2, pl.ds(0, m_q), :], cw_sems, 2, right),
            rdma(comm_cw.at[1, pl.ds(m_q, m_q), :],
                 comm_cw.at[2, pl.ds(m_q, m_q), :], cw_sems, 3, right),
        ]
        ccw = [
            rdma(own_ccw, comm_ccw.at[0], ccw_sems, 0, left),
            rdma(comm_ccw.at[0], comm_ccw.at[1], ccw_sems, 1, left),
            rdma(comm_ccw.at[1, pl.ds(0, m_q), :],
                 comm_ccw.at[2, pl.ds(0, m_q), :], ccw_sems, 2, left),
            rdma(comm_ccw.at[1, pl.ds(m_q, m_q), :],
                 comm_ccw.at[2, pl.ds(m_q, m_q), :], ccw_sems, 3, left),
        ]
        cw[0].start()
        ccw[0].start()

        amaxes = []
        subs = []
        main_rows = []

        def do_sub(chunk, row_start):
            i = len(subs)
            slot = i % 4
            if i >= 4:
                subs[i - 4].wait()
            acc = jax.lax.dot_general(
                chunk, w_ref[...],
                dimension_numbers=(((1,), (0,)), ((), ())),
                preferred_element_type=jnp.float32,
            )
            acc = jnp.maximum(acc, 0.0)
            stage[slot, :, :] = acc
            st = pltpu.make_async_copy(
                stage.at[slot],
                out_ref.at[pl.ds(row_start, m_q), :],
                store_sems.at[slot],
            )
            st.start()
            subs.append(st)
            amaxes.append(jnp.max(acc))

        def do_block(chunk, row_start):
            i = len(subs)
            s0, s1 = i % 4, (i + 1) % 4
            if i >= 4:
                subs[i - 4].wait()
            if i + 1 >= 4:
                subs[i - 3].wait()
            acc = jax.lax.dot_general(
                chunk, w_ref[...],
                dimension_numbers=(((1,), (0,)), ((), ())),
                preferred_element_type=jnp.float32,
            )
            acc = jnp.maximum(acc, 0.0)
            stage[s0, :, :] = acc[:m_q, :]
            stage[s1, :, :] = acc[m_q:, :]
            for j, s in enumerate((s0, s1)):
                st = pltpu.make_async_copy(
                    stage.at[s],
                    out_ref.at[pl.ds(row_start + j * m_q, m_q), :],
                    store_sems.at[s],
                )
                st.start()
                subs.append(st)
                main_rows.append(row_start + j * m_q)
            amaxes.append(jnp.max(acc))

        do_block(own_cw[:, :], me * m_loc)
        do_block(own_ccw[:, :], me * m_loc + m_half)

        cw[0].wait_recv()
        ccw[0].wait_recv()
        cw[1].start()
        ccw[1].start()
        o_cw = lax.rem(me + N_DEV - 1, N_DEV)
        o_ccw = lax.rem(me + 1, N_DEV)
        do_block(comm_cw[0], o_cw * m_loc)
        do_block(comm_ccw[0], o_ccw * m_loc + m_half)

        cw[1].wait_recv()
        ccw[1].wait_recv()
        for r in (cw[2], cw[3], ccw[2], ccw[3]):
            r.start()
        o_cw = lax.rem(me + N_DEV - 2, N_DEV)
        o_ccw = lax.rem(me + 2, N_DEV)
        do_block(comm_cw[1], o_cw * m_loc)
        do_block(comm_ccw[1], o_ccw * m_loc + m_half)

        o_cw = lax.rem(me + N_DEV - 3, N_DEV)
        o_ccw = lax.rem(me + 3, N_DEV)
        tail_rows = []

        def tail_sub(chunk, row_start):
            slot = len(tail_rows)
            subs[8 + slot].wait()
            acc = jax.lax.dot_general(
                chunk, w_ref[...],
                dimension_numbers=(((1,), (0,)), ((), ())),
                preferred_element_type=jnp.float32,
            )
            acc = jnp.maximum(acc, 0.0)
            stage[slot, :, :] = acc
            amaxes.append(jnp.max(acc))
            tail_rows.append(row_start)

        cw[2].wait_recv()
        tail_sub(comm_cw[2, pl.ds(0, m_q), :], o_cw * m_loc)
        ccw[2].wait_recv()
        tail_sub(comm_ccw[2, pl.ds(0, m_q), :], o_ccw * m_loc + m_half)
        cw[3].wait_recv()
        tail_sub(comm_cw[2, pl.ds(m_q, m_q), :], o_cw * m_loc + m_q)
        ccw[3].wait_recv()
        tail_sub(comm_ccw[2, pl.ds(m_q, m_q), :], o_ccw * m_loc + m_half + m_q)

        for r in cw + ccw:
            r.wait_send()

        local_amax = functools.reduce(jnp.maximum, amaxes)
        amax_box[...] = jnp.full((8, 128), local_amax, jnp.float32)
        a_rdmas = []
        for off in range(1, N_DEV):
            peer = lax.rem(me + off, N_DEV)
            r = pltpu.make_async_remote_copy(
                src_ref=amax_box, dst_ref=amax_recv.at[off - 1],
                send_sem=amax_send_sems.at[off - 1],
                recv_sem=amax_recv_sems.at[off - 1],
                device_id=(peer,), device_id_type=pl.DeviceIdType.MESH,
            )
            r.start()
            a_rdmas.append(r)

        g = local_amax
        for r in a_rdmas:
            r.wait_recv()
        for j in range(N_DEV - 1):
            g = jnp.maximum(g, jnp.max(amax_recv[j]))

        scale = g / 448.0
        inv = 448.0 / g

        def quant_dequant(v):
            vs = jnp.minimum(v * inv, 448.0)
            u = lax.bitcast_convert_type(vs, jnp.uint32)
            r = (u + 0x7FFFF + ((u >> 20) & 1)) & jnp.uint32(0xFFF00000)
            q = lax.bitcast_convert_type(r, jnp.float32)
            return q * scale

        def make_load(j):
            return pltpu.make_async_copy(
                out_ref.at[pl.ds(main_rows[j], m_q), :], stage.at[j % 4],
                load_sems.at[j % 4],
            )

        n_ep = n_sub - 4
        tail_stores = []
        for t in range(4):
            stage[t, :, :] = quant_dequant(stage[t, :, :])
            st = pltpu.make_async_copy(
                stage.at[t], out_ref.at[pl.ds(tail_rows[t], m_q), :],
                store_sems.at[t],
            )
            st.start()
            tail_stores.append(st)

        loads = {}
        for j in range(2):
            tail_stores[j].wait()
            loads[j] = make_load(j)
            loads[j].start()
        ep_stores = list(tail_stores[2:])
        for j in range(n_ep):
            slot = j % 4
            rows = pl.ds(main_rows[j], m_q)
            ep_stores[j].wait()
            if j + 2 < n_ep:
                loads[j + 2] = make_load(j + 2)
                loads[j + 2].start()
            loads[j].wait()
            stage[slot, :, :] = quant_dequant(stage[slot, :, :])
            st = pltpu.make_async_copy(
                stage.at[slot], out_ref.at[rows, :], store_sems.at[slot]
            )
            st.start()
            ep_stores.append(st)
        ep_stores[-2].wait()
        ep_stores[-1].wait()

        for r in a_rdmas:
            r.wait_send()

    return pl.pallas_call(
        body,
        out_shape=jax.ShapeDtypeStruct((m_glob, n_loc), jnp.float32),
        in_specs=[
            pl.BlockSpec(memory_space=pl.ANY),
            pl.BlockSpec(memory_space=pltpu.VMEM),
        ],
        out_specs=pl.BlockSpec(memory_space=pl.ANY),
        scratch_shapes=[
            pltpu.VMEM((m_half, k), jnp.bfloat16),
            pltpu.VMEM((m_half, k), jnp.bfloat16),
            pltpu.VMEM((n_hops, m_half, k), jnp.bfloat16),
            pltpu.VMEM((n_hops, m_half, k), jnp.bfloat16),
            pltpu.VMEM((4, m_q, n_loc), jnp.float32),
            pltpu.SemaphoreType.DMA((4,)),
            pltpu.SemaphoreType.DMA((4,)),
            pltpu.SemaphoreType.DMA((4,)),
            pltpu.SemaphoreType.DMA((4,)),
            pltpu.SemaphoreType.DMA((2,)),
            pltpu.SemaphoreType.DMA((4,)),
            pltpu.SemaphoreType.DMA((4,)),
            pltpu.VMEM((8, 128), jnp.float32),
            pltpu.VMEM((N_DEV - 1, 8, 128), jnp.float32),
            pltpu.SemaphoreType.DMA((N_DEV - 1,)),
            pltpu.SemaphoreType.DMA((N_DEV - 1,)),
        ],
        compiler_params=pltpu.CompilerParams(
            collective_id=0,
            vmem_limit_bytes=100 * 1024 * 1024,
        ),
    )(x, w_mat)
